# baseline (speedup 1.0000x reference)
import functools

import jax
import jax.numpy as jnp
from jax import lax
from jax.experimental import pallas as pl
from jax.experimental.pallas import tpu as pltpu
from jax.experimental.pallas import tpu_sc as plsc

R = 0.0625
R2 = R * R
K = 32
N = 16384
F = 128
H1 = 64
OUT = 128
S = 4096
BC = 64
CC = 128
NW = 32
GCH = 128
GW = 128

_INF = float("inf")


def _fps_body(px_ref, py_ref, pz_ref, sel_ref, cx_ref, cy_ref, cz_ref):
    px = px_ref[...]
    py = py_ref[...]
    pz = pz_ref[...]
    fi = (lax.broadcasted_iota(jnp.int32, (128, 128), 0) * 128
          + lax.broadcasted_iota(jnp.int32, (128, 128), 1))
    fo = (lax.broadcasted_iota(jnp.int32, (32, 128), 0) * 128
          + lax.broadcasted_iota(jnp.int32, (32, 128), 1))

    def extract(last):
        eq = fi == last
        lx = jnp.max(jnp.where(eq, px, -_INF))
        ly = jnp.max(jnp.where(eq, py, -_INF))
        lz = jnp.max(jnp.where(eq, pz, -_INF))
        return lx, ly, lz

    def body(i, st):
        dmin, sel, cx, cy, cz, last = st
        lx, ly, lz = extract(last)
        dx = px - lx
        dy = py - ly
        dz = pz - lz
        d = (dx * dx + dz * dz) + dy * dy
        dmin = jnp.minimum(dmin, d)
        m = jnp.max(dmin)
        nxt = jnp.min(jnp.where(dmin == m, fi, 2 ** 30))
        prev = fo == (i - 1)
        cx = jnp.where(prev, lx, cx)
        cy = jnp.where(prev, ly, cy)
        cz = jnp.where(prev, lz, cz)
        sel = jnp.where(fo == i, nxt, sel)
        return (dmin, sel, cx, cy, cz, nxt)

    init = (jnp.full((128, 128), _INF, jnp.float32),
            jnp.zeros((32, 128), jnp.int32),
            jnp.zeros((32, 128), jnp.float32),
            jnp.zeros((32, 128), jnp.float32),
            jnp.zeros((32, 128), jnp.float32),
            jnp.int32(0))
    _, sel, cx, cy, cz, last = lax.fori_loop(1, S, body, init)
    lx, ly, lz = extract(last)
    fin = fo == (S - 1)
    cx_ref[...] = jnp.where(fin, lx, cx)
    cy_ref[...] = jnp.where(fin, ly, cy)
    cz_ref[...] = jnp.where(fin, lz, cz)
    sel_ref[...] = sel


def _run_fps(px, py, pz):
    out = (jax.ShapeDtypeStruct((32, 128), jnp.int32),
           jax.ShapeDtypeStruct((32, 128), jnp.float32),
           jax.ShapeDtypeStruct((32, 128), jnp.float32),
           jax.ShapeDtypeStruct((32, 128), jnp.float32))
    return pl.pallas_call(_fps_body, out_shape=out)(px, py, pz)


def _prep_body(x_ref, pxc_ref, pyc_ref, pzc_ref, w1a_ref, wbx_ref, wby_ref,
               wbz_ref, b1_ref, a_ref):
    a = jnp.dot(x_ref[...], w1a_ref[...], preferred_element_type=jnp.float32)
    a = a + pxc_ref[...] * wbx_ref[...]
    a = a + pyc_ref[...] * wby_ref[...]
    a = a + pzc_ref[...] * wbz_ref[...]
    a = a + b1_ref[...]
    a_ref[...] = jnp.concatenate(
        [a, jnp.zeros((a.shape[0], GW - H1), jnp.float32)], axis=1)


_RB = 2048


def _run_prep(x, pxc, pyc, pzc, w1a, wbx, wby, wbz, b1row):
    grid = N // _RB
    xspec = pl.BlockSpec((_RB, F), lambda i: (i, 0))
    colspec = pl.BlockSpec((_RB, 1), lambda i: (i, 0))
    waspec = pl.BlockSpec((F, H1), lambda i: (0, 0))
    wrow = pl.BlockSpec((1, H1), lambda i: (0, 0))
    ospec = pl.BlockSpec((_RB, GW), lambda i: (i, 0))
    return pl.pallas_call(
        _prep_body,
        grid=(grid,),
        in_specs=[xspec, colspec, colspec, colspec, waspec, wrow, wrow,
                  wrow, wrow],
        out_specs=ospec,
        out_shape=jax.ShapeDtypeStruct((N, GW), jnp.float32),
    )(x, pxc, pyc, pzc, w1a, wbx, wby, wbz, b1row)


def _nbr_body(cx_ref, cy_ref, cz_ref, px_ref, py_ref, pz_ref,
              nbr_ref, vld_ref, d2_ref):
    cx = cx_ref[...]
    cy = cy_ref[...]
    cz = cz_ref[...]
    px = px_ref[...]
    py = py_ref[...]
    pz = pz_ref[...]
    c2 = (cx * cx + cz * cz) + cy * cy
    p2 = (px * px + pz * pz) + py * py
    cbx = cx.astype(jnp.bfloat16).astype(jnp.float32)
    cby = cy.astype(jnp.bfloat16).astype(jnp.float32)
    cbz = cz.astype(jnp.bfloat16).astype(jnp.float32)
    pbx = px.astype(jnp.bfloat16).astype(jnp.float32)
    pby = py.astype(jnp.bfloat16).astype(jnp.float32)
    pbz = pz.astype(jnp.bfloat16).astype(jnp.float32)
    mm = (cbx * pbx + cby * pby) + cbz * pbz
    d2 = (c2 + p2) - 2.0 * mm
    d2_ref[...] = jnp.where(d2 < R2, d2, _INF)
    lane = lax.broadcasted_iota(jnp.int32, (BC, N), 1)
    kio = lax.broadcasted_iota(jnp.int32, (BC, K), 1)

    def body(k, st):
        nbr, vld, thr, thri = st
        d2m = d2_ref[...]
        elig = (d2m > thr) | ((d2m == thr) & (lane > thri))
        cand = jnp.where(elig, d2m, _INF)
        m = jnp.min(cand, axis=1, keepdims=True)
        amin = jnp.min(jnp.where(cand == m, lane, 2 ** 30), axis=1,
                       keepdims=True)
        ok = (m < R2).astype(jnp.int32)
        sel = kio == k
        nbr = jnp.where(sel, amin, nbr)
        vld = jnp.where(sel, ok, vld)
        return (nbr, vld, m, amin)

    nbr, vld, _, _ = lax.fori_loop(
        0, K, body,
        (jnp.zeros((BC, K), jnp.int32), jnp.zeros((BC, K), jnp.int32),
         jnp.full((BC, 1), -_INF, jnp.float32),
         jnp.full((BC, 1), -1, jnp.int32)))
    nbr_ref[...] = nbr
    vld_ref[...] = vld


def _run_nbr(cxc, cyc, czc, pxr, pyr, pzr):
    grid = S // BC
    cspec = pl.BlockSpec((BC, 1), lambda i: (i, 0))
    pspec = pl.BlockSpec((1, N), lambda i: (0, 0))
    ospec = pl.BlockSpec((BC, K), lambda i: (i, 0))
    return pl.pallas_call(
        _nbr_body,
        grid=(grid,),
        in_specs=[cspec, cspec, cspec, pspec, pspec, pspec],
        out_specs=[ospec, ospec],
        out_shape=[jax.ShapeDtypeStruct((S, K), jnp.int32),
                   jax.ShapeDtypeStruct((S, K), jnp.int32)],
        scratch_shapes=[pltpu.VMEM((BC, N), jnp.float32)],
    )(cxc, cyc, czc, pxr, pyr, pzr)


def _sc_gather(a, idxflat):
    mesh = plsc.VectorSubcoreMesh(core_axis_name="c", subcore_axis_name="s")
    rows_per_w = (S * K) // NW
    steps = rows_per_w // GCH

    @functools.partial(
        pl.kernel,
        mesh=mesh,
        out_type=jax.ShapeDtypeStruct((S * K, GW), jnp.float32),
        scratch_types=[
            pltpu.VMEM((GCH,), jnp.int32),
            pltpu.VMEM((GCH, GW), jnp.float32),
            pltpu.SemaphoreType.DMA,
        ],
    )
    def gather_k(a_hbm, idx_hbm, out_hbm, idx_v, rows_v, sem):
        wid = lax.axis_index("s") * 2 + lax.axis_index("c")
        base = wid * rows_per_w

        def step(j, carry):
            off = base + j * GCH
            pltpu.sync_copy(idx_hbm.at[pl.ds(off, GCH)], idx_v)
            pltpu.async_copy(a_hbm.at[idx_v], rows_v, sem).wait()
            pltpu.sync_copy(rows_v, out_hbm.at[pl.ds(off, GCH)])
            return carry

        lax.fori_loop(0, steps, step, jnp.int32(0))

    return gather_k(a, idxflat)


def _conv_body(g_ref, cxe_ref, cye_ref, cze_ref, wbx_ref, wby_ref, wbz_ref,
               w2_ref, b2_ref, vldf_ref, vld_ref, out_ref):
    w = (cxe_ref[...] * wbx_ref[...] + cye_ref[...] * wby_ref[...]
         + cze_ref[...] * wbz_ref[...])
    h1 = jnp.maximum(g_ref[:, :H1] - w, 0.0)
    h2 = jnp.dot(h1, w2_ref[...],
                 preferred_element_type=jnp.float32) + b2_ref[...]
    pen = jnp.where(vldf_ref[...] != 0, 0.0, -_INF)
    h3 = (h2 + pen).reshape(CC, K, OUT)
    o = jnp.max(h3, axis=1)
    rowany = jnp.max(vld_ref[...], axis=1, keepdims=True)
    out_ref[...] = jnp.where(rowany != 0, o, 0.0)


def _run_conv(g, cxe, cye, cze, wbx, wby, wbz, W2, b2row, vldf, vld):
    grid = S // CC
    gspec = pl.BlockSpec((CC * K, GW), lambda i: (i, 0))
    espec = pl.BlockSpec((CC * K, 1), lambda i: (i, 0))
    wrow = pl.BlockSpec((1, H1), lambda i: (0, 0))
    w2spec = pl.BlockSpec((H1, OUT), lambda i: (0, 0))
    b2spec = pl.BlockSpec((1, OUT), lambda i: (0, 0))
    vspec = pl.BlockSpec((CC, K), lambda i: (i, 0))
    ospec = pl.BlockSpec((CC, OUT), lambda i: (i, 0))
    return pl.pallas_call(
        _conv_body,
        grid=(grid,),
        in_specs=[gspec, espec, espec, espec, wrow, wrow, wrow, w2spec,
                  b2spec, espec, vspec],
        out_specs=ospec,
        out_shape=jax.ShapeDtypeStruct((S, OUT), jnp.float32),
    )(g, cxe, cye, cze, wbx, wby, wbz, W2, b2row, vldf, vld)


def kernel(x, pos, batch, W1, b1, W2, b2):
    px = pos[:, 0].reshape(128, 128)
    py = pos[:, 1].reshape(128, 128)
    pz = pos[:, 2].reshape(128, 128)

    sel, cx, cy, cz = _run_fps(px, py, pz)
    idx = sel.reshape(S)
    cxc = cx.reshape(S, 1)
    cyc = cy.reshape(S, 1)
    czc = cz.reshape(S, 1)
    centers = jnp.concatenate([cxc, cyc, czc], axis=1)

    w1a = W1[:F]
    wbx = W1[F].reshape(1, H1)
    wby = W1[F + 1].reshape(1, H1)
    wbz = W1[F + 2].reshape(1, H1)
    b1row = b1.reshape(1, H1)
    a = _run_prep(x, pos[:, 0].reshape(N, 1), pos[:, 1].reshape(N, 1),
                  pos[:, 2].reshape(N, 1), w1a, wbx, wby, wbz, b1row)

    nbr, vld = _run_nbr(cxc, cyc, czc, pos[:, 0].reshape(1, N),
                        pos[:, 1].reshape(1, N), pos[:, 2].reshape(1, N))

    g = _sc_gather(a, nbr.reshape(S * K))

    cxe = jnp.repeat(cxc, K, axis=0)
    cye = jnp.repeat(cyc, K, axis=0)
    cze = jnp.repeat(czc, K, axis=0)
    vldf = vld.reshape(S * K, 1)
    out = _run_conv(g, cxe, cye, cze, wbx, wby, wbz, W2,
                    b2.reshape(1, OUT), vldf, vld)
    return (out, centers, batch[idx])


# speedup vs baseline: 1.1178x; 1.1178x over previous
import functools

import jax
import jax.numpy as jnp
from jax import lax
from jax.experimental import pallas as pl
from jax.experimental.pallas import tpu as pltpu
from jax.experimental.pallas import tpu_sc as plsc

R = 0.0625
R2 = R * R
K = 32
N = 16384
F = 128
H1 = 64
OUT = 128
S = 4096
BC = 64
CC = 128
NW = 32
GCH = 128
GW = 128

_INF = float("inf")


def _fps_body(px_ref, py_ref, pz_ref, sel_ref, cx_ref, cy_ref, cz_ref):
    px = px_ref[...]
    py = py_ref[...]
    pz = pz_ref[...]
    fi = (lax.broadcasted_iota(jnp.int32, (128, 128), 0) * 128
          + lax.broadcasted_iota(jnp.int32, (128, 128), 1))
    fo = (lax.broadcasted_iota(jnp.int32, (32, 128), 0) * 128
          + lax.broadcasted_iota(jnp.int32, (32, 128), 1))

    def extract(last):
        eq = fi == last
        lx = jnp.max(jnp.where(eq, px, -_INF))
        ly = jnp.max(jnp.where(eq, py, -_INF))
        lz = jnp.max(jnp.where(eq, pz, -_INF))
        return lx, ly, lz

    def body(i, st):
        dmin, sel, cx, cy, cz, last = st
        lx, ly, lz = extract(last)
        dx = px - lx
        dy = py - ly
        dz = pz - lz
        d = (dx * dx + dz * dz) + dy * dy
        dmin = jnp.minimum(dmin, d)
        m = jnp.max(dmin)
        nxt = jnp.min(jnp.where(dmin == m, fi, 2 ** 30))
        prev = fo == (i - 1)
        cx = jnp.where(prev, lx, cx)
        cy = jnp.where(prev, ly, cy)
        cz = jnp.where(prev, lz, cz)
        sel = jnp.where(fo == i, nxt, sel)
        return (dmin, sel, cx, cy, cz, nxt)

    init = (jnp.full((128, 128), _INF, jnp.float32),
            jnp.zeros((32, 128), jnp.int32),
            jnp.zeros((32, 128), jnp.float32),
            jnp.zeros((32, 128), jnp.float32),
            jnp.zeros((32, 128), jnp.float32),
            jnp.int32(0))
    _, sel, cx, cy, cz, last = lax.fori_loop(1, S, body, init)
    lx, ly, lz = extract(last)
    fin = fo == (S - 1)
    cx_ref[...] = jnp.where(fin, lx, cx)
    cy_ref[...] = jnp.where(fin, ly, cy)
    cz_ref[...] = jnp.where(fin, lz, cz)
    sel_ref[...] = sel


def _run_fps(px, py, pz):
    out = (jax.ShapeDtypeStruct((32, 128), jnp.int32),
           jax.ShapeDtypeStruct((32, 128), jnp.float32),
           jax.ShapeDtypeStruct((32, 128), jnp.float32),
           jax.ShapeDtypeStruct((32, 128), jnp.float32))
    return pl.pallas_call(_fps_body, out_shape=out)(px, py, pz)


def _prep_body(x_ref, pxc_ref, pyc_ref, pzc_ref, w1a_ref, wbx_ref, wby_ref,
               wbz_ref, b1_ref, a_ref):
    a = jnp.dot(x_ref[...], w1a_ref[...], preferred_element_type=jnp.float32)
    a = a + pxc_ref[...] * wbx_ref[...]
    a = a + pyc_ref[...] * wby_ref[...]
    a = a + pzc_ref[...] * wbz_ref[...]
    a = a + b1_ref[...]
    a_ref[...] = jnp.concatenate(
        [a, jnp.zeros((a.shape[0], GW - H1), jnp.float32)], axis=1)


_RB = 2048


def _run_prep(x, pxc, pyc, pzc, w1a, wbx, wby, wbz, b1row):
    grid = N // _RB
    xspec = pl.BlockSpec((_RB, F), lambda i: (i, 0))
    colspec = pl.BlockSpec((_RB, 1), lambda i: (i, 0))
    waspec = pl.BlockSpec((F, H1), lambda i: (0, 0))
    wrow = pl.BlockSpec((1, H1), lambda i: (0, 0))
    ospec = pl.BlockSpec((_RB, GW), lambda i: (i, 0))
    return pl.pallas_call(
        _prep_body,
        grid=(grid,),
        in_specs=[xspec, colspec, colspec, colspec, waspec, wrow, wrow,
                  wrow, wrow],
        out_specs=ospec,
        out_shape=jax.ShapeDtypeStruct((N, GW), jnp.float32),
    )(x, pxc, pyc, pzc, w1a, wbx, wby, wbz, b1row)


def _nbr_body(cx_ref, cy_ref, cz_ref, px_ref, py_ref, pz_ref,
              nbr_ref, vld_ref, d2_ref):
    cx = cx_ref[...]
    cy = cy_ref[...]
    cz = cz_ref[...]
    px = px_ref[...]
    py = py_ref[...]
    pz = pz_ref[...]
    c2 = (cx * cx + cz * cz) + cy * cy
    p2 = (px * px + pz * pz) + py * py
    cbx = cx.astype(jnp.bfloat16).astype(jnp.float32)
    cby = cy.astype(jnp.bfloat16).astype(jnp.float32)
    cbz = cz.astype(jnp.bfloat16).astype(jnp.float32)
    pbx = px.astype(jnp.bfloat16).astype(jnp.float32)
    pby = py.astype(jnp.bfloat16).astype(jnp.float32)
    pbz = pz.astype(jnp.bfloat16).astype(jnp.float32)
    mm = (cbx * pbx + cby * pby) + cbz * pbz
    d2 = (c2 + p2) - 2.0 * mm
    d2_ref[...] = jnp.where(d2 < R2, d2, _INF)
    lane = lax.broadcasted_iota(jnp.int32, (BC, N), 1)
    kio = lax.broadcasted_iota(jnp.int32, (BC, K), 1)

    def body(k, st):
        nbr, vld = st
        d2m = d2_ref[...]
        m = jnp.min(d2m, axis=1, keepdims=True)
        amin = jnp.min(jnp.where(d2m == m, lane, 2 ** 30), axis=1,
                       keepdims=True)
        ok = (m < R2).astype(jnp.int32)
        sel = kio == k
        nbr = jnp.where(sel, amin, nbr)
        vld = jnp.where(sel, ok, vld)
        d2_ref[...] = jnp.where(lane == amin, _INF, d2m)
        return (nbr, vld)

    nbr, vld = lax.fori_loop(
        0, K, body,
        (jnp.zeros((BC, K), jnp.int32), jnp.zeros((BC, K), jnp.int32)))
    nbr_ref[...] = nbr
    vld_ref[...] = vld


def _run_nbr(cxc, cyc, czc, pxr, pyr, pzr):
    grid = S // BC
    cspec = pl.BlockSpec((BC, 1), lambda i: (i, 0))
    pspec = pl.BlockSpec((1, N), lambda i: (0, 0))
    ospec = pl.BlockSpec((BC, K), lambda i: (i, 0))
    return pl.pallas_call(
        _nbr_body,
        grid=(grid,),
        in_specs=[cspec, cspec, cspec, pspec, pspec, pspec],
        out_specs=[ospec, ospec],
        out_shape=[jax.ShapeDtypeStruct((S, K), jnp.int32),
                   jax.ShapeDtypeStruct((S, K), jnp.int32)],
        scratch_shapes=[pltpu.VMEM((BC, N), jnp.float32)],
    )(cxc, cyc, czc, pxr, pyr, pzr)


def _sc_gather(a, idxflat):
    mesh = plsc.VectorSubcoreMesh(core_axis_name="c", subcore_axis_name="s")
    rows_per_w = (S * K) // NW
    steps = rows_per_w // GCH

    @functools.partial(
        pl.kernel,
        mesh=mesh,
        out_type=jax.ShapeDtypeStruct((S * K, GW), jnp.float32),
        scratch_types=[
            pltpu.VMEM((GCH,), jnp.int32),
            pltpu.VMEM((GCH, GW), jnp.float32),
            pltpu.SemaphoreType.DMA,
        ],
    )
    def gather_k(a_hbm, idx_hbm, out_hbm, idx_v, rows_v, sem):
        wid = lax.axis_index("s") * 2 + lax.axis_index("c")
        base = wid * rows_per_w

        def step(j, carry):
            off = base + j * GCH
            pltpu.sync_copy(idx_hbm.at[pl.ds(off, GCH)], idx_v)
            pltpu.async_copy(a_hbm.at[idx_v], rows_v, sem).wait()
            pltpu.sync_copy(rows_v, out_hbm.at[pl.ds(off, GCH)])
            return carry

        lax.fori_loop(0, steps, step, jnp.int32(0))

    return gather_k(a, idxflat)


def _conv_body(g_ref, cxe_ref, cye_ref, cze_ref, wbx_ref, wby_ref, wbz_ref,
               w2_ref, b2_ref, vldf_ref, vld_ref, out_ref):
    w = (cxe_ref[...] * wbx_ref[...] + cye_ref[...] * wby_ref[...]
         + cze_ref[...] * wbz_ref[...])
    h1 = jnp.maximum(g_ref[:, :H1] - w, 0.0)
    h2 = jnp.dot(h1, w2_ref[...],
                 preferred_element_type=jnp.float32) + b2_ref[...]
    pen = jnp.where(vldf_ref[...] != 0, 0.0, -_INF)
    h3 = (h2 + pen).reshape(CC, K, OUT)
    o = jnp.max(h3, axis=1)
    rowany = jnp.max(vld_ref[...], axis=1, keepdims=True)
    out_ref[...] = jnp.where(rowany != 0, o, 0.0)


def _run_conv(g, cxe, cye, cze, wbx, wby, wbz, W2, b2row, vldf, vld):
    grid = S // CC
    gspec = pl.BlockSpec((CC * K, GW), lambda i: (i, 0))
    espec = pl.BlockSpec((CC * K, 1), lambda i: (i, 0))
    wrow = pl.BlockSpec((1, H1), lambda i: (0, 0))
    w2spec = pl.BlockSpec((H1, OUT), lambda i: (0, 0))
    b2spec = pl.BlockSpec((1, OUT), lambda i: (0, 0))
    vspec = pl.BlockSpec((CC, K), lambda i: (i, 0))
    ospec = pl.BlockSpec((CC, OUT), lambda i: (i, 0))
    return pl.pallas_call(
        _conv_body,
        grid=(grid,),
        in_specs=[gspec, espec, espec, espec, wrow, wrow, wrow, w2spec,
                  b2spec, espec, vspec],
        out_specs=ospec,
        out_shape=jax.ShapeDtypeStruct((S, OUT), jnp.float32),
    )(g, cxe, cye, cze, wbx, wby, wbz, W2, b2row, vldf, vld)


def kernel(x, pos, batch, W1, b1, W2, b2):
    px = pos[:, 0].reshape(128, 128)
    py = pos[:, 1].reshape(128, 128)
    pz = pos[:, 2].reshape(128, 128)

    sel, cx, cy, cz = _run_fps(px, py, pz)
    idx = sel.reshape(S)
    cxc = cx.reshape(S, 1)
    cyc = cy.reshape(S, 1)
    czc = cz.reshape(S, 1)
    centers = jnp.concatenate([cxc, cyc, czc], axis=1)

    w1a = W1[:F]
    wbx = W1[F].reshape(1, H1)
    wby = W1[F + 1].reshape(1, H1)
    wbz = W1[F + 2].reshape(1, H1)
    b1row = b1.reshape(1, H1)
    a = _run_prep(x, pos[:, 0].reshape(N, 1), pos[:, 1].reshape(N, 1),
                  pos[:, 2].reshape(N, 1), w1a, wbx, wby, wbz, b1row)

    nbr, vld = _run_nbr(cxc, cyc, czc, pos[:, 0].reshape(1, N),
                        pos[:, 1].reshape(1, N), pos[:, 2].reshape(1, N))

    g = _sc_gather(a, nbr.reshape(S * K))

    cxe = jnp.repeat(cxc, K, axis=0)
    cye = jnp.repeat(cyc, K, axis=0)
    cze = jnp.repeat(czc, K, axis=0)
    vldf = vld.reshape(S * K, 1)
    out = _run_conv(g, cxe, cye, cze, wbx, wby, wbz, W2,
                    b2.reshape(1, OUT), vldf, vld)
    return (out, centers, batch[idx])
